# baseline (speedup 1.0000x reference)
import functools

import jax
import jax.numpy as jnp
from jax import lax
from jax.experimental import pallas as pl
from jax.experimental.pallas import tpu as pltpu
from jax.experimental.pallas import tpu_sc as plsc

_NUM_ELEMENTS = 119
_FEAT_DIM = 92
_D_MODEL = 256
_N_ATOMS = 100000

_VPAD = 128
_FPAD = 128

_NC = 2
_NS = 16
_NW = _NC * _NS
_L = 16

_CHUNK = 80
_NUM_CHUNKS = _N_ATOMS // _CHUNK
_NBUF = 2
_NI = -(-_NUM_CHUNKS // _NW)


def _fuse_body(t_ref, w_ref, b_ref, o_ref):
    o_ref[...] = (
        jnp.dot(t_ref[...], w_ref[...], preferred_element_type=jnp.float32)
        + b_ref[...]
    )


def _fused_table(table, W, b):
    tp = jnp.zeros((_VPAD, _FPAD), jnp.float32).at[:_NUM_ELEMENTS, :_FEAT_DIM].set(table)
    wp = jnp.zeros((_FPAD, _D_MODEL), jnp.float32).at[:_FEAT_DIM].set(W)
    return pl.pallas_call(
        _fuse_body,
        out_shape=jax.ShapeDtypeStruct((_VPAD, _D_MODEL), jnp.float32),
    )(tp, wp, b.reshape(1, _D_MODEL))


_mesh = plsc.VectorSubcoreMesh(
    core_axis_name="c", subcore_axis_name="s", num_cores=_NC, num_subcores=_NS
)


@functools.partial(
    pl.kernel,
    out_type=jax.ShapeDtypeStruct((_N_ATOMS * _D_MODEL,), jnp.float32),
    mesh=_mesh,
    compiler_params=pltpu.CompilerParams(needs_layout_passes=False),
    scratch_types=[
        pltpu.VMEM((_NBUF, _CHUNK), jnp.int32),
        pltpu.VMEM((_CHUNK * _D_MODEL,), jnp.float32),
        pltpu.VMEM((_CHUNK * _D_MODEL,), jnp.float32),
        pltpu.VMEM((_VPAD * _D_MODEL,), jnp.float32),
    ]
    + [pltpu.SemaphoreType.DMA] * (2 * _NBUF),
)
def _gather(x_hbm, fused_hbm, out_hbm, idx_v, rows0_v, rows1_v, fused_v, *sems):
    rows_bufs = (rows0_v, rows1_v)
    isems = sems[0:_NBUF]
    wsems = sems[_NBUF : 2 * _NBUF]
    wid = lax.axis_index("s") * _NC + lax.axis_index("c")

    pltpu.sync_copy(fused_hbm, fused_v)

    def cid(i):
        return wid + i * _NW

    def start_idx(i, p):
        pltpu.async_copy(
            x_hbm.at[pl.ds(cid(i) * _CHUNK, _CHUNK)], idx_v.at[p], isems[p]
        )

    lane = lax.iota(jnp.int32, _L)
    row_off = lane * _D_MODEL
    _GRAN = 8

    def compute_chunk(p):
        rows_flat = rows_bufs[p]
        for g in range(_CHUNK // _L):
            iv = idx_v[p, pl.ds(g * _L, _L)]
            src_row = iv * _D_MODEL
            dst_row = row_off + (g * _L * _D_MODEL)

            @plsc.parallel_loop(0, _L)
            def blk(s):
                grot = jnp.bitwise_and(lane + s, _L - 1) * _GRAN
                src_g = src_row + grot
                dst_g = dst_row + grot
                for h in range(_D_MODEL // (_GRAN * _L)):
                    c0 = h * _GRAN * _L
                    vals = [
                        plsc.load_gather(fused_v, [src_g + (c0 + e)])
                        for e in range(_GRAN)
                    ]
                    for e in range(_GRAN):
                        plsc.store_scatter(rows_flat, [dst_g + (c0 + e)], vals[e])

    for p in range(_NBUF):
        start_idx(p, p)

    def body(k, carry):
        for p in range(_NBUF):
            i = _NBUF * k + p

            @pl.when(cid(i) < _NUM_CHUNKS)
            def _process():
                pltpu.make_async_copy(
                    x_hbm.at[pl.ds(0, _CHUNK)], idx_v.at[p], isems[p]
                ).wait()

                @pl.when(k >= 1)
                def _drain_prev():
                    pltpu.make_async_copy(
                        rows_bufs[p], out_hbm.at[pl.ds(0, _CHUNK * _D_MODEL)],
                        wsems[p],
                    ).wait()

                compute_chunk(p)

                pltpu.async_copy(
                    rows_bufs[p],
                    out_hbm.at[pl.ds(cid(i) * (_CHUNK * _D_MODEL), _CHUNK * _D_MODEL)],
                    wsems[p],
                )

                @pl.when(cid(i + _NBUF) < _NUM_CHUNKS)
                def _prefetch():
                    start_idx(i + _NBUF, p)

        return carry

    lax.fori_loop(0, _NI // _NBUF, body, 0)

    for p in range(_NBUF):
        pltpu.make_async_copy(
            rows_bufs[p], out_hbm.at[pl.ds(0, _CHUNK * _D_MODEL)], wsems[p]
        ).wait()


def kernel(x, table, W, b):
    fused = _fused_table(table, W, b)
    out_flat = _gather(x, fused.reshape(_VPAD * _D_MODEL))
    return out_flat.reshape(_N_ATOMS, _D_MODEL)


# speedup vs baseline: 1.6152x; 1.6152x over previous
import functools

import jax
import jax.numpy as jnp
from jax import lax
from jax.experimental import pallas as pl
from jax.experimental.pallas import tpu as pltpu
from jax.experimental.pallas import tpu_sc as plsc

_NUM_ELEMENTS = 119
_FEAT_DIM = 92
_D_MODEL = 256
_N_ATOMS = 100000

_VPAD = 128
_FPAD = 128

_NC = 2
_NS = 16
_NW = _NC * _NS
_L = 16

_CHUNK = 80
_NUM_CHUNKS = _N_ATOMS // _CHUNK
_NBUF = 2
_NI = -(-_NUM_CHUNKS // _NW)


def _fuse_body(t_ref, w_ref, b_ref, o_ref):
    o_ref[...] = (
        jnp.dot(t_ref[...], w_ref[...], preferred_element_type=jnp.float32)
        + b_ref[...]
    )


def _fused_table(table, W, b):
    tp = jnp.zeros((_VPAD, _FPAD), jnp.float32).at[:_NUM_ELEMENTS, :_FEAT_DIM].set(table)
    wp = jnp.zeros((_FPAD, _D_MODEL), jnp.float32).at[:_FEAT_DIM].set(W)
    return pl.pallas_call(
        _fuse_body,
        out_shape=jax.ShapeDtypeStruct((_VPAD, _D_MODEL), jnp.float32),
    )(tp, wp, b.reshape(1, _D_MODEL))


_mesh = plsc.VectorSubcoreMesh(
    core_axis_name="c", subcore_axis_name="s", num_cores=_NC, num_subcores=_NS
)


@functools.partial(
    pl.kernel,
    out_type=jax.ShapeDtypeStruct((_N_ATOMS * _D_MODEL,), jnp.float32),
    mesh=_mesh,
    compiler_params=pltpu.CompilerParams(needs_layout_passes=False),
    scratch_types=[
        pltpu.SMEM((_NBUF, _CHUNK), jnp.int32),
        pltpu.VMEM((_CHUNK * _D_MODEL,), jnp.float32),
        pltpu.VMEM((_CHUNK * _D_MODEL,), jnp.float32),
        pltpu.VMEM((_VPAD * _D_MODEL,), jnp.float32),
        pltpu.VMEM_SHARED((_N_ATOMS,), jnp.int32),
    ]
    + [pltpu.SemaphoreType.DMA] * (2 * _NBUF),
)
def _gather(x_hbm, fused_hbm, out_hbm, idx_v, rows0_v, rows1_v, fused_v, x_sh, *sems):
    rows_bufs = (rows0_v, rows1_v)
    isems = sems[0:_NBUF]
    wsems = sems[_NBUF : 2 * _NBUF]
    wid = lax.axis_index("s") * _NC + lax.axis_index("c")

    @pl.when(lax.axis_index("s") == 0)
    def _stage_x():
        pltpu.sync_copy(x_hbm, x_sh)

    pltpu.sync_copy(fused_hbm, fused_v)
    plsc.subcore_barrier()

    def cid(i):
        return wid + i * _NW

    def start_idx(i, p):
        pltpu.async_copy(
            x_sh.at[pl.ds(cid(i) * _CHUNK, _CHUNK)], idx_v.at[p], isems[p]
        )

    def compute_chunk(p):
        rows_flat = rows_bufs[p]

        @plsc.parallel_loop(0, _CHUNK, unroll=2)
        def row(r):
            src0 = idx_v[p, r] * _D_MODEL
            dst0 = r * _D_MODEL
            for c in range(_D_MODEL // _L):
                rows_flat[pl.ds(dst0 + c * _L, _L)] = fused_v[
                    pl.ds(src0 + c * _L, _L)
                ]

    for p in range(_NBUF):
        start_idx(p, p)

    def body(k, carry):
        for p in range(_NBUF):
            i = _NBUF * k + p

            @pl.when(cid(i) < _NUM_CHUNKS)
            def _process():
                pltpu.make_async_copy(
                    x_sh.at[pl.ds(0, _CHUNK)], idx_v.at[p], isems[p]
                ).wait()

                @pl.when(k >= 1)
                def _drain_prev():
                    pltpu.make_async_copy(
                        rows_bufs[p], out_hbm.at[pl.ds(0, _CHUNK * _D_MODEL)],
                        wsems[p],
                    ).wait()

                compute_chunk(p)

                pltpu.async_copy(
                    rows_bufs[p],
                    out_hbm.at[pl.ds(cid(i) * (_CHUNK * _D_MODEL), _CHUNK * _D_MODEL)],
                    wsems[p],
                )

                @pl.when(cid(i + _NBUF) < _NUM_CHUNKS)
                def _prefetch():
                    start_idx(i + _NBUF, p)

        return carry

    lax.fori_loop(0, _NI // _NBUF, body, 0)

    for p in range(_NBUF):
        pltpu.make_async_copy(
            rows_bufs[p], out_hbm.at[pl.ds(0, _CHUNK * _D_MODEL)], wsems[p]
        ).wait()


def kernel(x, table, W, b):
    fused = _fused_table(table, W, b)
    out_flat = _gather(x, fused.reshape(_VPAD * _D_MODEL))
    return out_flat.reshape(_N_ATOMS, _D_MODEL)
